# baseline (speedup 1.0000x reference)
import functools

import jax
import jax.numpy as jnp
from jax import lax
from jax.experimental import pallas as pl
from jax.experimental.pallas import tpu as pltpu
from jax.experimental.pallas import tpu_sc as plsc

BATCH = 4096
N_FIELDS = 26
N_DENSE = 13
VOCAB = 100000
EMBED_DIM = 64
OUT_COLS = N_FIELDS * EMBED_DIM + N_DENSE

NC = 2
NS = 16
NW = NC * NS
ROWS_PER_W = BATCH // NW
LANES = 16


def _body(sidx_hbm, dense_hbm, tables_hbm, out_hbm,
          idx_v, fidx_v, rows_v, dense_v, sem, wsem):
    wid = lax.axis_index("s") * NC + lax.axis_index("c")
    base = wid * ROWS_PER_W

    pltpu.sync_copy(sidx_hbm.at[pl.ds(base * N_FIELDS, ROWS_PER_W * N_FIELDS)],
                    idx_v)
    pltpu.sync_copy(dense_hbm.at[pl.ds(base, ROWS_PER_W)], dense_v)
    pltpu.sync_copy(dense_v,
                    out_hbm.at[pl.ds(base, ROWS_PER_W),
                               pl.ds(N_FIELDS * EMBED_DIM, N_DENSE)])

    lane = lax.iota(jnp.int32, LANES)
    for f in range(N_FIELDS):
        for g in range(ROWS_PER_W // LANES):
            pos = lane * N_FIELDS + ((g * LANES) * N_FIELDS + f)
            vals = plsc.load_gather(idx_v, [pos])
            fidx_v[pl.ds(g * LANES, LANES)] = vals + (f * VOCAB)
        pltpu.async_copy(tables_hbm.at[fidx_v], rows_v, sem).wait()
        pltpu.async_copy(rows_v,
                         out_hbm.at[pl.ds(base, ROWS_PER_W),
                                    pl.ds(f * EMBED_DIM, EMBED_DIM)],
                         wsem).wait()


@functools.partial(jax.jit, static_argnums=())
def _sc_embed(sidx_flat, dense_values, tables_flat):
    mesh = plsc.VectorSubcoreMesh(core_axis_name="c", subcore_axis_name="s")
    fn = pl.kernel(
        _body,
        mesh=mesh,
        compiler_params=pltpu.CompilerParams(use_tc_tiling_on_sc=False,
                                             needs_layout_passes=False),
        out_type=jax.ShapeDtypeStruct((BATCH, OUT_COLS), jnp.float32),
        scratch_types=[
            pltpu.VMEM((ROWS_PER_W * N_FIELDS,), jnp.int32),
            pltpu.VMEM((ROWS_PER_W,), jnp.int32),
            pltpu.VMEM((ROWS_PER_W, EMBED_DIM), jnp.float32),
            pltpu.VMEM((ROWS_PER_W, N_DENSE), jnp.float32),
            pltpu.SemaphoreType.DMA,
            pltpu.SemaphoreType.DMA,
        ],
    )
    return fn(sidx_flat, dense_values, tables_flat)


def kernel(sparse_indices, dense_values, tables):
    sidx_flat = sparse_indices.reshape(-1)
    tables_flat = tables.reshape(N_FIELDS * VOCAB, EMBED_DIM)
    return _sc_embed(sidx_flat, dense_values, tables_flat)


# speedup vs baseline: 1.9491x; 1.9491x over previous
import jax
import jax.numpy as jnp
from jax import lax
from jax.experimental import pallas as pl
from jax.experimental.pallas import tpu as pltpu
from jax.experimental.pallas import tpu_sc as plsc

BATCH = 4096
N_FIELDS = 26
N_DENSE = 13
VOCAB = 100000
EMBED = 64
LANES = 16

NC = 2
NS = 16

W = 256
NCHUNK = VOCAB // W + 1
NFULL = NCHUNK - 1
TAIL0 = (VOCAB // 128) * 128
TAIL_MAIN0 = NFULL * W
NROWS = N_FIELDS * BATCH
FLUSH = 128
DUMP0 = NROWS
OUT3_ROWS = NROWS + 32 * FLUSH
HBUF = 400


def _body(sidx_hbm, tablesT_hbm, tail_hbm, out_hbm,
          vidx_v, sorted_v, hist_v, cur_v, off_s,
          buf0, buf1, stag_v, oidx_v, sem0, sem1, wsem):
    wid = lax.axis_index("s") * NC + lax.axis_index("c")

    @pl.when(wid < N_FIELDS)
    def _active():
        f = wid
        lane = lax.iota(jnp.int32, LANES)
        lane0 = lane == 0
        zeros16 = jnp.zeros((LANES,), jnp.int32)
        ones16 = jnp.ones((LANES,), jnp.int32)

        pltpu.sync_copy(
            sidx_hbm.at[pl.ds(pl.multiple_of(f * BATCH, BATCH), BATCH)],
            vidx_v)

        def _issue(c, buf, sem):
            col0 = pl.multiple_of(c * W, W)
            pltpu.async_copy(tablesT_hbm.at[f, :, pl.ds(col0, W)], buf, sem)

        def _wait(buf, sem):
            pltpu.make_async_copy(tablesT_hbm.at[f, :, pl.ds(0, W)],
                                  buf, sem).wait()

        _issue(0, buf0, sem0)
        _issue(1, buf1, sem1)

        for t in range(HBUF // LANES):
            hist_v[pl.ds(t * LANES, LANES)] = zeros16

        def _hist(t, carry):
            v16 = plsc.load_gather(vidx_v, [t * LANES + lane])
            plsc.addupdate_scatter(hist_v, [v16 >> 8], ones16)
            return carry
        lax.fori_loop(0, BATCH // LANES, _hist, 0)

        carry = jnp.int32(0)
        for t in range(HBUF // LANES):
            h16 = hist_v[pl.ds(t * LANES, LANES)]
            incl = plsc.cumsum(h16)
            excl = incl - h16 + carry
            cur_v[pl.ds(t * LANES, LANES)] = excl
            for i in range(LANES):
                off_s[t * LANES + i] = excl[i]
            carry = carry + incl[LANES - 1]

        def _place(t, carry):
            kv = t * LANES + lane
            v16 = plsc.load_gather(vidx_v, [kv])
            c16 = v16 >> 8
            rec16 = (kv << 8) | (v16 & (W - 1))
            dup16, _last = plsc.scan_count(c16)
            base16 = plsc.load_gather(cur_v, [c16])
            plsc.store_scatter(sorted_v, [base16 + dup16 - 1], rec16)
            plsc.addupdate_scatter(cur_v, [c16], ones16)
            return carry
        lax.fori_loop(0, BATCH // LANES, _place, 0)

        dump = DUMP0 + wid * FLUSH

        def _reset_oidx():
            for t in range(FLUSH // LANES):
                oidx_v[pl.ds(t * LANES, LANES)] = dump + t * LANES + lane
        _reset_oidx()

        def _flush():
            pltpu.async_copy(stag_v, out_hbm.at[oidx_v], wsem).wait()
            _reset_oidx()

        def _process(c, buf, j):
            n0 = off_s[c]
            n1 = off_s[c + 1]
            ngroups = (n1 - n0 + LANES - 1) >> 4

            def _group(g, j):
                k0 = n0 + g * LANES
                kidx = jnp.minimum(k0 + lane, n1 - 1)
                rec16 = plsc.load_gather(sorted_v, [kidx])
                for i in range(LANES):
                    valid = (k0 + i) < n1
                    do_flush = jnp.logical_and(valid, j == FLUSH)

                    @pl.when(do_flush)
                    def _():
                        _flush()
                    j = lax.select(do_flush, 0, j)
                    rec = rec16[i]
                    col = rec & (W - 1)
                    b = rec >> 8
                    j16 = jnp.full((LANES,), j, jnp.int32)

                    @pl.when(valid)
                    def _():
                        plsc.store_scatter(
                            oidx_v, [j16],
                            jnp.full((LANES,), f * BATCH + b, jnp.int32),
                            mask=lane0)
                        cols = jnp.full((LANES,), col, jnp.int32)
                        for t in range(EMBED // LANES):
                            vals = plsc.load_gather(
                                buf, [lane + t * LANES, cols])
                            plsc.store_scatter(stag_v,
                                               [j16, lane + t * LANES], vals)
                    j = j + lax.select(valid, 1, 0)
                return j
            return lax.fori_loop(0, ngroups, _group, j)

        def _pair(i, j):
            c0 = 2 * i
            _wait(buf0, sem0)
            j = _process(c0, buf0, j)

            @pl.when(c0 + 2 < NFULL)
            def _():
                _issue(c0 + 2, buf0, sem0)
            _wait(buf1, sem1)
            j = _process(c0 + 1, buf1, j)

            @pl.when(c0 + 3 < NFULL)
            def _():
                _issue(c0 + 3, buf1, sem1)
            return j

        j = lax.fori_loop(0, NFULL // 2, _pair, 0)

        pltpu.sync_copy(tablesT_hbm.at[f, :, pl.ds(TAIL_MAIN0, 128)],
                        buf0.at[:, pl.ds(0, 128)])
        pltpu.sync_copy(tail_hbm.at[f], buf0.at[:, pl.ds(128, 128)])
        j = _process(NFULL, buf0, j)
        _flush()


@jax.jit
def _sc_embed(sidxT, tablesT, tailT):
    mesh = plsc.VectorSubcoreMesh(core_axis_name="c", subcore_axis_name="s")
    fn = pl.kernel(
        _body,
        mesh=mesh,
        compiler_params=pltpu.CompilerParams(needs_layout_passes=False),
        out_type=jax.ShapeDtypeStruct((OUT3_ROWS, 2 * EMBED), jnp.float32),
        scratch_types=[
            pltpu.VMEM((BATCH,), jnp.int32),
            pltpu.VMEM((BATCH,), jnp.int32),
            pltpu.VMEM((HBUF,), jnp.int32),
            pltpu.VMEM((HBUF,), jnp.int32),
            pltpu.SMEM((HBUF,), jnp.int32),
            pltpu.VMEM((EMBED, W), jnp.float32),
            pltpu.VMEM((EMBED, W), jnp.float32),
            pltpu.VMEM((FLUSH, 2 * EMBED), jnp.float32),
            pltpu.VMEM((FLUSH,), jnp.int32),
            pltpu.SemaphoreType.DMA,
            pltpu.SemaphoreType.DMA,
            pltpu.SemaphoreType.DMA,
        ],
    )
    return fn(sidxT, tablesT, tailT)


def kernel(sparse_indices, dense_values, tables):
    tablesT = tables.transpose(0, 2, 1)
    tailT = tables[:, TAIL0:, :].transpose(0, 2, 1)
    tailT = jnp.pad(tailT, ((0, 0), (0, 0), (0, 128 - (VOCAB - TAIL0))))
    sidxT = sparse_indices.T.reshape(-1)
    out3 = _sc_embed(sidxT, tablesT, tailT)
    emb = out3[:NROWS, :EMBED].reshape(N_FIELDS, BATCH, EMBED)
    emb = emb.transpose(1, 0, 2).reshape(BATCH, N_FIELDS * EMBED)
    return jnp.concatenate([emb, dense_values], axis=1)


# speedup vs baseline: 2.4161x; 1.2396x over previous
import jax
import jax.numpy as jnp
from jax import lax
from jax.experimental import pallas as pl
from jax.experimental.pallas import tpu as pltpu
from jax.experimental.pallas import tpu_sc as plsc

BATCH = 4096
N_FIELDS = 26
N_DENSE = 13
VOCAB = 100000
EMBED = 64
LANES = 16

NC = 2
NS = 16

W = 256
NCHUNK = VOCAB // W + 1
NFULL = NCHUNK - 1
TAIL0 = (VOCAB // 128) * 128
TAIL_MAIN0 = NFULL * W
NROWS = N_FIELDS * BATCH
FLUSH = 128
DUMP0 = NROWS
OUT3_ROWS = NROWS + 32 * FLUSH
HBUF = 400


def _body(sidx_hbm, tablesT_hbm, tail_hbm, out_hbm,
          vidx_v, sorted_v, hist_v, cur_v, off_s,
          buf0, buf1, buf2, stag_v, oidx_v, sem0, sem1, sem2, wsem):
    wid = lax.axis_index("s") * NC + lax.axis_index("c")

    @pl.when(wid < N_FIELDS)
    def _active():
        f = wid
        lane = lax.iota(jnp.int32, LANES)
        zeros16 = jnp.zeros((LANES,), jnp.int32)
        ones16 = jnp.ones((LANES,), jnp.int32)

        pltpu.sync_copy(
            sidx_hbm.at[pl.ds(pl.multiple_of(f * BATCH, BATCH), BATCH)],
            vidx_v)

        def _issue(c, buf, sem):
            col0 = pl.multiple_of(c * W, W)
            pltpu.async_copy(tablesT_hbm.at[f, :, pl.ds(col0, W)], buf, sem)

        def _wait(buf, sem):
            pltpu.make_async_copy(tablesT_hbm.at[f, :, pl.ds(0, W)],
                                  buf, sem).wait()

        _issue(0, buf0, sem0)
        _issue(1, buf1, sem1)
        _issue(2, buf2, sem2)

        for t in range(HBUF // LANES):
            hist_v[pl.ds(t * LANES, LANES)] = zeros16

        def _hist(t, carry):
            v16 = plsc.load_gather(vidx_v, [t * LANES + lane])
            plsc.addupdate_scatter(hist_v, [v16 >> 8], ones16)
            return carry
        lax.fori_loop(0, BATCH // LANES, _hist, 0)

        carry = jnp.int32(0)
        for t in range(HBUF // LANES):
            h16 = hist_v[pl.ds(t * LANES, LANES)]
            incl = plsc.cumsum(h16)
            excl = incl - h16 + carry
            cur_v[pl.ds(t * LANES, LANES)] = excl
            for i in range(LANES):
                off_s[t * LANES + i] = excl[i]
            carry = carry + incl[LANES - 1]

        def _place(t, carry):
            kv = t * LANES + lane
            v16 = plsc.load_gather(vidx_v, [kv])
            c16 = v16 >> 8
            rec16 = (kv << 8) | (v16 & (W - 1))
            dup16, _last = plsc.scan_count(c16)
            base16 = plsc.load_gather(cur_v, [c16])
            plsc.store_scatter(sorted_v, [base16 + dup16 - 1], rec16)
            plsc.addupdate_scatter(cur_v, [c16], ones16)
            return carry
        lax.fori_loop(0, BATCH // LANES, _place, 0)

        dump = DUMP0 + wid * FLUSH
        dump_lane = dump + lane

        def _reset_oidx():
            for t in range(FLUSH // LANES):
                oidx_v[pl.ds(t * LANES, LANES)] = dump + t * LANES + lane
        _reset_oidx()

        def _flush():
            pltpu.async_copy(stag_v, out_hbm.at[oidx_v], wsem).wait()
            _reset_oidx()

        def _process(c, buf, j):
            n0 = off_s[c]
            n1 = off_s[c + 1]
            ngroups = (n1 - n0 + LANES - 1) >> 4

            def _group(g, j):
                @pl.when(j == FLUSH)
                def _():
                    _flush()
                j = lax.select(j == FLUSH, 0, j)
                kv = n0 + g * LANES + lane
                kidx = jnp.minimum(kv, n1 - 1)
                rec16 = plsc.load_gather(sorted_v, [kidx])
                rowid16 = jnp.where(kv < n1, f * BATCH + (rec16 >> 8),
                                    dump_lane)
                plsc.store_scatter(oidx_v, [j + lane], rowid16)
                for i in range(LANES):
                    col = rec16[i] & (W - 1)
                    cols = jnp.full((LANES,), col, jnp.int32)
                    j16 = jnp.full((LANES,), j + i, jnp.int32)
                    for t in range(EMBED // LANES):
                        vals = plsc.load_gather(buf, [lane + t * LANES, cols])
                        plsc.store_scatter(stag_v, [j16, lane + t * LANES],
                                           vals)
                return j + LANES
            return lax.fori_loop(0, ngroups, _group, j)

        def _trip(i, j):
            c0 = 3 * i
            _wait(buf0, sem0)
            j = _process(c0, buf0, j)

            @pl.when(c0 + 3 < NFULL)
            def _():
                _issue(c0 + 3, buf0, sem0)
            _wait(buf1, sem1)
            j = _process(c0 + 1, buf1, j)

            @pl.when(c0 + 4 < NFULL)
            def _():
                _issue(c0 + 4, buf1, sem1)
            _wait(buf2, sem2)
            j = _process(c0 + 2, buf2, j)

            @pl.when(c0 + 5 < NFULL)
            def _():
                _issue(c0 + 5, buf2, sem2)
            return j

        j = lax.fori_loop(0, NFULL // 3, _trip, 0)

        pltpu.sync_copy(tablesT_hbm.at[f, :, pl.ds(TAIL_MAIN0, 128)],
                        buf0.at[:, pl.ds(0, 128)])
        pltpu.sync_copy(tail_hbm.at[f], buf0.at[:, pl.ds(128, 128)])
        j = _process(NFULL, buf0, j)
        _flush()


@jax.jit
def _sc_embed(sidxT, tablesT, tailT):
    mesh = plsc.VectorSubcoreMesh(core_axis_name="c", subcore_axis_name="s")
    fn = pl.kernel(
        _body,
        mesh=mesh,
        compiler_params=pltpu.CompilerParams(needs_layout_passes=False),
        out_type=jax.ShapeDtypeStruct((OUT3_ROWS, 2 * EMBED), jnp.float32),
        scratch_types=[
            pltpu.VMEM((BATCH,), jnp.int32),
            pltpu.VMEM((BATCH,), jnp.int32),
            pltpu.VMEM((HBUF,), jnp.int32),
            pltpu.VMEM((HBUF,), jnp.int32),
            pltpu.SMEM((HBUF,), jnp.int32),
            pltpu.VMEM((EMBED, W), jnp.float32),
            pltpu.VMEM((EMBED, W), jnp.float32),
            pltpu.VMEM((EMBED, W), jnp.float32),
            pltpu.VMEM((FLUSH, 2 * EMBED), jnp.float32),
            pltpu.VMEM((FLUSH,), jnp.int32),
            pltpu.SemaphoreType.DMA,
            pltpu.SemaphoreType.DMA,
            pltpu.SemaphoreType.DMA,
            pltpu.SemaphoreType.DMA,
        ],
    )
    return fn(sidxT, tablesT, tailT)


def kernel(sparse_indices, dense_values, tables):
    tablesT = tables.transpose(0, 2, 1)
    tailT = tables[:, TAIL0:, :].transpose(0, 2, 1)
    tailT = jnp.pad(tailT, ((0, 0), (0, 0), (0, 128 - (VOCAB - TAIL0))))
    sidxT = sparse_indices.T.reshape(-1)
    out3 = _sc_embed(sidxT, tablesT, tailT)
    emb = out3[:NROWS, :EMBED].reshape(N_FIELDS, BATCH, EMBED)
    emb = emb.transpose(1, 0, 2).reshape(BATCH, N_FIELDS * EMBED)
    return jnp.concatenate([emb, dense_values], axis=1)
